# Initial kernel scaffold; baseline (speedup 1.0000x reference)
import jax
import jax.numpy as jnp
from jax.experimental import pallas as pl


def kernel(x, edge_index, W1, b1, W2, b2, W3, b3):
    raise NotImplementedError("write your pallas kernel here")


# speedup vs baseline: 3.8477x; 3.8477x over previous
import functools

import jax
import jax.numpy as jnp
from jax import lax
from jax.experimental import pallas as pl
from jax.experimental.pallas import tpu as pltpu
from jax.experimental.pallas import tpu_sc as plsc

N = 10000
E = 320000
D = 128

NC = 2
NS = 16
CHUNK = 128
EP = 327680
NCHUNKS = EP // CHUNK
CPT = NCHUNKS // (NC * NS)
HALF = CPT // 2
ZROWS = 632
AGG_ROWS = NS * ZROWS
WB = 624
WB_LAST = N - (NS - 1) * WB


def _sc_agg_body(h_hbm, src_hbm, dst_hbm, out_hbm,
                 src_all, dst_all, rows0, rows1, agg_sh, semg0, semg1):
    c = lax.axis_index("c")
    s = lax.axis_index("s")
    wid = c * NS + s

    def zbody(i, carry):
        for j in range(D // 16):
            rows0[i, pl.ds(j * 16, 16)] = jnp.zeros((16,), jnp.float32)
        return carry
    lax.fori_loop(0, CHUNK, zbody, 0)

    zbase = s * ZROWS
    for k in range(ZROWS // CHUNK):
        pltpu.sync_copy(rows0, agg_sh.at[pl.ds(zbase + k * CHUNK, CHUNK)])
    rem = ZROWS % CHUNK
    if rem:
        pltpu.sync_copy(rows0.at[pl.ds(0, rem)],
                        agg_sh.at[pl.ds(zbase + (ZROWS // CHUNK) * CHUNK, rem)])
    plsc.subcore_barrier()

    def body(gg, carry):
        g0 = 2 * gg
        g1 = g0 + 1
        pltpu.async_copy(h_hbm.at[src_all.at[g1]], rows1, semg1)
        pltpu.make_async_copy(h_hbm.at[src_all.at[g0]], rows0, semg0).wait()
        pltpu.sync_copy(rows0, agg_sh.at[dst_all.at[g0]], add=True)

        @pl.when(gg + 1 < HALF // 2)
        def _():
            pltpu.async_copy(h_hbm.at[src_all.at[g0 + 2]], rows0, semg0)

        pltpu.make_async_copy(h_hbm.at[src_all.at[g1]], rows1, semg1).wait()
        pltpu.sync_copy(rows1, agg_sh.at[dst_all.at[g1]], add=True)
        return carry

    for half in range(CPT // HALF):
        base = wid * CPT + half * HALF
        pltpu.sync_copy(src_hbm.at[pl.ds(base, HALF)], src_all)
        pltpu.sync_copy(dst_hbm.at[pl.ds(base, HALF)], dst_all)
        pltpu.async_copy(h_hbm.at[src_all.at[0]], rows0, semg0)
        lax.fori_loop(0, HALF // 2, body, 0)

    plsc.subcore_barrier()

    @pl.when(s < NS - 1)
    def _():
        pltpu.sync_copy(agg_sh.at[pl.ds(s * WB, WB)],
                        out_hbm.at[pl.ds(c * N + s * WB, WB)])

    @pl.when(s == NS - 1)
    def _():
        pltpu.sync_copy(agg_sh.at[pl.ds((NS - 1) * WB, WB_LAST)],
                        out_hbm.at[pl.ds(c * N + (NS - 1) * WB, WB_LAST)])


def _sc_agg(h, src2, dst2):
    mesh = plsc.VectorSubcoreMesh(core_axis_name="c", subcore_axis_name="s",
                                  num_cores=NC, num_subcores=NS)
    return pl.kernel(
        _sc_agg_body,
        out_type=jax.ShapeDtypeStruct((2 * N, D), jnp.float32),
        mesh=mesh,
        scratch_types=[
            pltpu.VMEM((HALF, CHUNK), jnp.int32),
            pltpu.VMEM((HALF, CHUNK), jnp.int32),
            pltpu.VMEM((CHUNK, D), jnp.float32),
            pltpu.VMEM((CHUNK, D), jnp.float32),
            pltpu.VMEM_SHARED((AGG_ROWS, D), jnp.float32),
            pltpu.SemaphoreType.DMA,
            pltpu.SemaphoreType.DMA,
        ],
    )(h, src2, dst2)


def _tc_mlp_body(relu, x_ref, a0_ref, a1_ref, w_ref, b_ref, o_ref):
    t = x_ref[...] + a0_ref[...] + a1_ref[...]
    y = jnp.dot(t, w_ref[...], preferred_element_type=jnp.float32) + b_ref[...]
    if relu:
        y = jnp.maximum(y, 0.0)
    o_ref[...] = y


def _tc_mlp(h, agg, w, b, relu):
    blk = 1000
    grid = N // blk
    return pl.pallas_call(
        functools.partial(_tc_mlp_body, relu),
        grid=(grid,),
        in_specs=[
            pl.BlockSpec((blk, D), lambda i: (i, 0)),
            pl.BlockSpec((blk, D), lambda i: (i, 0)),
            pl.BlockSpec((blk, D), lambda i: (i + grid, 0)),
            pl.BlockSpec((D, D), lambda i: (0, 0)),
            pl.BlockSpec((1, D), lambda i: (0, 0)),
        ],
        out_specs=pl.BlockSpec((blk, D), lambda i: (i, 0)),
        out_shape=jax.ShapeDtypeStruct((N, D), jnp.float32),
    )(h, agg, agg, w, b.reshape(1, D))


def kernel(x, edge_index, W1, b1, W2, b2, W3, b3):
    src = edge_index[0].astype(jnp.int32)
    dst = edge_index[1].astype(jnp.int32)
    pad = EP - E
    src2 = jnp.concatenate([src, jnp.zeros((pad,), jnp.int32)]).reshape(NCHUNKS, CHUNK)
    dst2 = jnp.concatenate([dst, jnp.full((pad,), N, jnp.int32)]).reshape(NCHUNKS, CHUNK)

    agg = _sc_agg(x, src2, dst2)
    h = _tc_mlp(x, agg, W1, b1, relu=True)
    agg = _sc_agg(h, src2, dst2)
    h = _tc_mlp(h, agg, W2, b2, relu=True)
    agg = _sc_agg(h, src2, dst2)
    w3p = jnp.zeros((D, D), jnp.float32).at[:, :40].set(W3)
    b3p = jnp.zeros((D,), jnp.float32).at[:40].set(b3)
    h = _tc_mlp(h, agg, w3p, b3p, relu=False)
    return h[:, :40]


# speedup vs baseline: 4.4167x; 1.1479x over previous
import functools

import jax
import jax.numpy as jnp
from jax import lax
from jax.experimental import pallas as pl
from jax.experimental.pallas import tpu as pltpu
from jax.experimental.pallas import tpu_sc as plsc

N = 10000
E = 320000
D = 128
COLH = 64

NC = 2
NS = 16
CHUNK = 128
EP = 327680
NCHUNKS = EP // CHUNK
CPT = NCHUNKS // NS
QCH = 40
NBUF = 8
ZROWS = 632
AGG_ROWS = NS * ZROWS
WB = 624
WB_LAST = N - (NS - 1) * WB


def _sc_agg_body(hcat, srcoff, dst2, out, *scr):
    src_st = scr[0]
    dst_st = scr[1]
    bufs = scr[2:2 + NBUF]
    agg_sh = scr[2 + NBUF]
    semg = scr[3 + NBUF:3 + 2 * NBUF]
    sems = scr[3 + 2 * NBUF:3 + 3 * NBUF]

    c = lax.axis_index("c")
    s = lax.axis_index("s")

    def zbody(i, carry):
        for j in range(COLH // 16):
            bufs[0][i, pl.ds(j * 16, 16)] = jnp.zeros((16,), jnp.float32)
        return carry
    lax.fori_loop(0, CHUNK, zbody, 0)

    zbase = s * ZROWS
    for k in range(ZROWS // CHUNK):
        pltpu.sync_copy(bufs[0], agg_sh.at[pl.ds(zbase + k * CHUNK, CHUNK)])
    rem = ZROWS % CHUNK
    if rem:
        pltpu.sync_copy(bufs[0].at[pl.ds(0, rem)],
                        agg_sh.at[pl.ds(zbase + (ZROWS // CHUNK) * CHUNK, rem)])
    plsc.subcore_barrier()

    def wait_gather(j, g):
        pltpu.make_async_copy(hcat.at[src_st.at[g]], bufs[j], semg[j]).wait()

    def wait_scatter(j):
        pltpu.make_async_copy(bufs[j], agg_sh.at[dst_st.at[0]], sems[j]).wait()

    cbase = s * CPT
    for q in range(CPT // QCH):
        if q > 0:
            for j in range(NBUF):
                wait_scatter(j)
        qb = cbase + q * QCH
        pltpu.sync_copy(srcoff.at[pl.ds(c * NCHUNKS + qb, QCH)], src_st)
        pltpu.sync_copy(dst2.at[pl.ds(qb, QCH)], dst_st)

        def qloop(gg, carry):
            base = gg * NBUF
            for j in range(NBUF):
                @pl.when(gg > 0)
                def _(j=j):
                    wait_scatter(j)
                pltpu.async_copy(hcat.at[src_st.at[base + j]], bufs[j], semg[j])
            for j in range(NBUF):
                wait_gather(j, base + j)
                pltpu.async_copy(bufs[j], agg_sh.at[dst_st.at[base + j]],
                                 sems[j], add=True)
            return carry

        lax.fori_loop(0, QCH // NBUF, qloop, 0)

    for j in range(NBUF):
        wait_scatter(j)

    plsc.subcore_barrier()

    @pl.when(s < NS - 1)
    def _():
        pltpu.sync_copy(agg_sh.at[pl.ds(s * WB, WB)],
                        out.at[pl.ds(c * N + s * WB, WB)])

    @pl.when(s == NS - 1)
    def _():
        pltpu.sync_copy(agg_sh.at[pl.ds((NS - 1) * WB, WB_LAST)],
                        out.at[pl.ds(c * N + (NS - 1) * WB, WB_LAST)])


def _sc_agg(hcat, srcoff, dst2):
    mesh = plsc.VectorSubcoreMesh(core_axis_name="c", subcore_axis_name="s",
                                  num_cores=NC, num_subcores=NS)
    return pl.kernel(
        _sc_agg_body,
        out_type=jax.ShapeDtypeStruct((2 * N, COLH), jnp.float32),
        mesh=mesh,
        compiler_params=pltpu.CompilerParams(use_tc_tiling_on_sc=False),
        scratch_types=[
            pltpu.VMEM((QCH, CHUNK), jnp.int32),
            pltpu.VMEM((QCH, CHUNK), jnp.int32),
        ] + [pltpu.VMEM((CHUNK, COLH), jnp.float32) for _ in range(NBUF)]
        + [pltpu.VMEM_SHARED((AGG_ROWS, COLH), jnp.float32)]
        + [pltpu.SemaphoreType.DMA for _ in range(2 * NBUF)],
    )(hcat, srcoff, dst2)


def _tc_mlp_body(relu, hl_ref, hr_ref, al_ref, ar_ref, w_ref, b_ref, o_ref):
    t = jnp.concatenate(
        [hl_ref[...] + al_ref[...], hr_ref[...] + ar_ref[...]], axis=1)
    y = jnp.dot(t, w_ref[0], preferred_element_type=jnp.float32) + b_ref[0]
    if relu:
        y = jnp.maximum(y, 0.0)
    o_ref[...] = y


def _tc_mlp(hcat, agg, w, b, relu):
    blk = 1000
    nb = N // blk
    return pl.pallas_call(
        functools.partial(_tc_mlp_body, relu),
        grid=(2 * nb,),
        in_specs=[
            pl.BlockSpec((blk, COLH), lambda i: (i % nb, 0)),
            pl.BlockSpec((blk, COLH), lambda i: (nb + i % nb, 0)),
            pl.BlockSpec((blk, COLH), lambda i: (i % nb, 0)),
            pl.BlockSpec((blk, COLH), lambda i: (nb + i % nb, 0)),
            pl.BlockSpec((1, D, COLH), lambda i: (i // nb, 0, 0)),
            pl.BlockSpec((1, 1, COLH), lambda i: (i // nb, 0, 0)),
        ],
        out_specs=pl.BlockSpec((blk, COLH), lambda i: (i, 0)),
        out_shape=jax.ShapeDtypeStruct((2 * N, COLH), jnp.float32),
    )(hcat, hcat, agg, agg,
      jnp.stack([w[:, :COLH], w[:, COLH:]]),
      b.reshape(2, 1, COLH))


def kernel(x, edge_index, W1, b1, W2, b2, W3, b3):
    src = edge_index[0].astype(jnp.int32)
    dst = edge_index[1].astype(jnp.int32)
    pad = EP - E
    src2 = jnp.concatenate([src, jnp.zeros((pad,), jnp.int32)]).reshape(NCHUNKS, CHUNK)
    dst2 = jnp.concatenate([dst, jnp.full((pad,), N, jnp.int32)]).reshape(NCHUNKS, CHUNK)
    srcoff = jnp.concatenate([src2, src2 + N])

    hcat = jnp.concatenate([x[:, :COLH], x[:, COLH:]], axis=0)
    agg = _sc_agg(hcat, srcoff, dst2)
    hcat = _tc_mlp(hcat, agg, W1, b1, relu=True)
    agg = _sc_agg(hcat, srcoff, dst2)
    hcat = _tc_mlp(hcat, agg, W2, b2, relu=True)
    agg = _sc_agg(hcat, srcoff, dst2)
    w3p = jnp.zeros((D, D), jnp.float32).at[:, :40].set(W3)
    b3p = jnp.zeros((D,), jnp.float32).at[:40].set(b3)
    hcat = _tc_mlp(hcat, agg, w3p, b3p, relu=False)
    return hcat[:N, :40]


# speedup vs baseline: 9.4567x; 2.1411x over previous
import functools

import jax
import jax.numpy as jnp
from jax import lax
from jax.experimental import pallas as pl
from jax.experimental.pallas import tpu as pltpu
from jax.experimental.pallas import tpu_sc as plsc

N = 10000
E = 320000
D = 128
COLH = 64

NC = 2
NS = 16
CHUNK = 256
EP = 327680
NCHUNKS = EP // CHUNK
CPT = NCHUNKS // NS
QCH = 20
NBUF = 4
ZCH = 128
ZROWS = 632
AGG_ROWS = NS * ZROWS
WB = 624
WB_LAST = N - (NS - 1) * WB


def _sc_agg_body(hcat, srcoff, dst2, out, *scr):
    src_st = scr[0]
    dst_st = scr[1]
    bufs = scr[2:2 + NBUF]
    agg_sh = scr[2 + NBUF]
    semg = scr[3 + NBUF:3 + 2 * NBUF]
    sems = scr[3 + 2 * NBUF:3 + 3 * NBUF]

    c = lax.axis_index("c")
    s = lax.axis_index("s")

    def zbody(i, carry):
        for j in range(COLH // 16):
            bufs[0][i, pl.ds(j * 16, 16)] = jnp.zeros((16,), jnp.float32)
        return carry
    lax.fori_loop(0, ZCH, zbody, 0)

    zbase = s * ZROWS
    for k in range(ZROWS // ZCH):
        pltpu.sync_copy(bufs[0].at[pl.ds(0, ZCH)],
                        agg_sh.at[pl.ds(zbase + k * ZCH, ZCH)])
    rem = ZROWS % ZCH
    if rem:
        pltpu.sync_copy(bufs[0].at[pl.ds(0, rem)],
                        agg_sh.at[pl.ds(zbase + (ZROWS // ZCH) * ZCH, rem)])
    plsc.subcore_barrier()

    def wait_gather(j, g):
        pltpu.make_async_copy(hcat.at[src_st.at[g]], bufs[j], semg[j]).wait()

    def wait_scatter(j):
        pltpu.make_async_copy(bufs[j], agg_sh.at[dst_st.at[0]], sems[j]).wait()

    cbase = s * CPT
    for q in range(CPT // QCH):
        if q > 0:
            for j in range(NBUF):
                wait_scatter(j)
        qb = cbase + q * QCH
        pltpu.sync_copy(srcoff.at[pl.ds(c * NCHUNKS + qb, QCH)], src_st)
        pltpu.sync_copy(dst2.at[pl.ds(qb, QCH)], dst_st)

        def qloop(gg, carry):
            base = gg * NBUF
            for j in range(NBUF):
                @pl.when(gg > 0)
                def _(j=j):
                    wait_scatter(j)
                pltpu.async_copy(hcat.at[src_st.at[base + j]], bufs[j], semg[j])
            for j in range(NBUF):
                wait_gather(j, base + j)
                pltpu.async_copy(bufs[j], agg_sh.at[dst_st.at[base + j]],
                                 sems[j], add=True)
            return carry

        lax.fori_loop(0, QCH // NBUF, qloop, 0)

    for j in range(NBUF):
        wait_scatter(j)

    plsc.subcore_barrier()

    @pl.when(s < NS - 1)
    def _():
        pltpu.sync_copy(agg_sh.at[pl.ds(s * WB, WB)],
                        out.at[pl.ds(c * N + s * WB, WB)])

    @pl.when(s == NS - 1)
    def _():
        pltpu.sync_copy(agg_sh.at[pl.ds((NS - 1) * WB, WB_LAST)],
                        out.at[pl.ds(c * N + (NS - 1) * WB, WB_LAST)])


def _sc_agg(hcat, srcoff, dst2):
    mesh = plsc.VectorSubcoreMesh(core_axis_name="c", subcore_axis_name="s",
                                  num_cores=NC, num_subcores=NS)
    return pl.kernel(
        _sc_agg_body,
        out_type=jax.ShapeDtypeStruct((2 * N, COLH), jnp.float32),
        mesh=mesh,
        compiler_params=pltpu.CompilerParams(use_tc_tiling_on_sc=False),
        scratch_types=[
            pltpu.VMEM((QCH, CHUNK), jnp.int32),
            pltpu.VMEM((QCH, CHUNK), jnp.int32),
        ] + [pltpu.VMEM((CHUNK, COLH), jnp.float32) for _ in range(NBUF)]
        + [pltpu.VMEM_SHARED((AGG_ROWS, COLH), jnp.float32)]
        + [pltpu.SemaphoreType.DMA for _ in range(2 * NBUF)],
    )(hcat, srcoff, dst2)


def _tc_mlp_body(relu, hl_ref, hr_ref, al_ref, ar_ref, w_ref, b_ref, o_ref):
    t = jnp.concatenate(
        [hl_ref[...] + al_ref[...], hr_ref[...] + ar_ref[...]], axis=1)
    y = jnp.dot(t, w_ref[0], preferred_element_type=jnp.float32) + b_ref[0]
    if relu:
        y = jnp.maximum(y, 0.0)
    o_ref[...] = y


def _tc_mlp(hcat, agg, w, b, relu):
    blk = 1000
    nb = N // blk
    return pl.pallas_call(
        functools.partial(_tc_mlp_body, relu),
        grid=(2 * nb,),
        in_specs=[
            pl.BlockSpec((blk, COLH), lambda i: (i % nb, 0)),
            pl.BlockSpec((blk, COLH), lambda i: (nb + i % nb, 0)),
            pl.BlockSpec((blk, COLH), lambda i: (i % nb, 0)),
            pl.BlockSpec((blk, COLH), lambda i: (nb + i % nb, 0)),
            pl.BlockSpec((1, D, COLH), lambda i: (i // nb, 0, 0)),
            pl.BlockSpec((1, 1, COLH), lambda i: (i // nb, 0, 0)),
        ],
        out_specs=pl.BlockSpec((blk, COLH), lambda i: (i, 0)),
        out_shape=jax.ShapeDtypeStruct((2 * N, COLH), jnp.float32),
    )(hcat, hcat, agg, agg,
      jnp.stack([w[:, :COLH], w[:, COLH:]]),
      b.reshape(2, 1, COLH))


def kernel(x, edge_index, W1, b1, W2, b2, W3, b3):
    src = edge_index[0].astype(jnp.int32)
    dst = edge_index[1].astype(jnp.int32)
    pad = EP - E
    src2 = jnp.concatenate([src, (jnp.arange(pad, dtype=jnp.int32) * 13) % N]
                           ).reshape(NCHUNKS, CHUNK)
    dst2 = jnp.concatenate([dst, jnp.full((pad,), N, jnp.int32)]).reshape(NCHUNKS, CHUNK)
    srcoff = jnp.concatenate([src2, src2 + N])

    hcat = jnp.concatenate([x[:, :COLH], x[:, COLH:]], axis=0)
    agg = _sc_agg(hcat, srcoff, dst2)
    hcat = _tc_mlp(hcat, agg, W1, b1, relu=True)
    agg = _sc_agg(hcat, srcoff, dst2)
    hcat = _tc_mlp(hcat, agg, W2, b2, relu=True)
    agg = _sc_agg(hcat, srcoff, dst2)
    w3p = jnp.zeros((D, D), jnp.float32).at[:, :40].set(W3)
    b3p = jnp.zeros((D,), jnp.float32).at[:40].set(b3)
    hcat = _tc_mlp(hcat, agg, w3p, b3p, relu=False)
    return hcat[:N, :40]
